# baseline (speedup 1.0000x reference)
import jax
import jax.numpy as jnp
from jax import lax
from jax.experimental import pallas as pl
from jax.experimental.pallas import tpu as pltpu
from jax.experimental.pallas import tpu_sc as plsc

SEQ = 2048
DIM = 256
BATCH = 4
NC = 2
NS = 16
NW = NC * NS
S_PER_W = SEQ // NW
NQ = BATCH // 2
LANES = 16
DCHUNKS = DIM // LANES


def _emb_body(x_hbm, tok_hbm, pos_hbm, out_hbm, idx_v, tok_v, pos_v,
              idx_sem, pos_sem, gat_sem, st_sem):
    wid = lax.axis_index("s") * NC + lax.axis_index("c")
    s0 = wid * S_PER_W

    idx_cp = pltpu.async_copy(x_hbm.at[wid], idx_v, idx_sem)
    pos_cp = pltpu.async_copy(pos_hbm.at[pl.ds(s0, S_PER_W)], pos_v, pos_sem)

    idx_cp.wait()
    gathers = [
        pltpu.async_copy(tok_hbm.at[idx_v.at[q]], tok_v.at[q], gat_sem)
        for q in range(NQ)
    ]
    pos_cp.wait()

    stores = []
    for q in range(NQ):
        gathers[q].wait()

        @plsc.parallel_loop(0, S_PER_W, unroll=4)
        def _add(r):
            for c in range(DCHUNKS):
                sl = pl.ds(c * LANES, LANES)
                p = pos_v[r, sl]
                plsc.addupdate(tok_v.at[q, r, sl], p)
                plsc.addupdate(tok_v.at[q, S_PER_W + r, sl], p)

        for i in range(2):
            b = 2 * q + i
            stores.append(pltpu.async_copy(
                tok_v.at[q, pl.ds(i * S_PER_W, S_PER_W)],
                out_hbm.at[pl.ds(b * SEQ + s0, S_PER_W)], st_sem))
    for st in stores:
        st.wait()


def kernel(x, token_table, pos_table):
    B, S = x.shape
    xr = (x.astype(jnp.int32)
          .reshape(NQ, 2, NW, S_PER_W)
          .transpose(2, 0, 1, 3)
          .reshape(NW, NQ, 2 * S_PER_W))
    call = pl.kernel(
        _emb_body,
        out_type=jax.ShapeDtypeStruct((B * S, DIM), jnp.float32),
        mesh=plsc.VectorSubcoreMesh(core_axis_name="c", subcore_axis_name="s"),
        scratch_types=[
            pltpu.VMEM((NQ, 2 * S_PER_W), jnp.int32),
            pltpu.VMEM((NQ, 2 * S_PER_W, DIM), jnp.float32),
            pltpu.VMEM((S_PER_W, DIM), jnp.float32),
            pltpu.SemaphoreType.DMA,
            pltpu.SemaphoreType.DMA,
            pltpu.SemaphoreType.DMA,
            pltpu.SemaphoreType.DMA,
        ],
    )
    out = call(xr, token_table, pos_table)
    return out.reshape(B, S, DIM)


# speedup vs baseline: 1.0119x; 1.0119x over previous
import jax
import jax.numpy as jnp
from jax import lax
from jax.experimental import pallas as pl
from jax.experimental.pallas import tpu as pltpu
from jax.experimental.pallas import tpu_sc as plsc

SEQ = 2048
DIM = 256
BATCH = 4
NC = 2
NS = 16
NW = NC * NS
S_PER_W = SEQ // NW
NQ = BATCH // 2
LANES = 16
DCHUNKS = DIM // LANES


def _emb_body(x_hbm, tok_hbm, pos_hbm, out_hbm, idx_v, tok_v, pos_v,
              idx_sem, pos_sem, gat_sem, st_sem):
    wid = lax.axis_index("s") * NC + lax.axis_index("c")
    s0 = wid * S_PER_W

    idx_cp = pltpu.async_copy(x_hbm.at[wid], idx_v, idx_sem)
    pos_cp = pltpu.async_copy(pos_hbm.at[pl.ds(s0, S_PER_W)], pos_v, pos_sem)

    idx_cp.wait()
    gathers = [
        pltpu.async_copy(tok_hbm.at[idx_v.at[q]], tok_v.at[q], gat_sem)
        for q in range(NQ)
    ]
    pos_cp.wait()

    stores = []
    for q in range(NQ):
        gathers[q].wait()
        for i in range(2):

            @plsc.parallel_loop(0, S_PER_W)
            def _add(r):
                for c in range(DCHUNKS):
                    sl = pl.ds(c * LANES, LANES)
                    plsc.addupdate(tok_v.at[q, i * S_PER_W + r, sl],
                                   pos_v[r, sl])

            stores.append(pltpu.async_copy(
                tok_v.at[q, pl.ds(i * S_PER_W, S_PER_W)],
                out_hbm.at[pl.ds((2 * q + i) * SEQ + s0, S_PER_W)], st_sem))
    for st in stores:
        st.wait()


def kernel(x, token_table, pos_table):
    B, S = x.shape
    xr = (x.astype(jnp.int32)
          .reshape(NQ, 2, NW, S_PER_W)
          .transpose(2, 0, 1, 3)
          .reshape(NW, NQ, 2 * S_PER_W))
    call = pl.kernel(
        _emb_body,
        out_type=jax.ShapeDtypeStruct((B * S, DIM), jnp.float32),
        mesh=plsc.VectorSubcoreMesh(core_axis_name="c", subcore_axis_name="s"),
        scratch_types=[
            pltpu.VMEM((NQ, 2 * S_PER_W), jnp.int32),
            pltpu.VMEM((NQ, 2 * S_PER_W, DIM), jnp.float32),
            pltpu.VMEM((S_PER_W, DIM), jnp.float32),
            pltpu.SemaphoreType.DMA,
            pltpu.SemaphoreType.DMA,
            pltpu.SemaphoreType.DMA,
            pltpu.SemaphoreType.DMA,
        ],
    )
    out = call(xr, token_table, pos_table)
    return out.reshape(B, S, DIM)


# speedup vs baseline: 1.0422x; 1.0300x over previous
import jax
import jax.numpy as jnp
from jax import lax
from jax.experimental import pallas as pl
from jax.experimental.pallas import tpu as pltpu
from jax.experimental.pallas import tpu_sc as plsc

SEQ = 2048
DIM = 256
BATCH = 4
NC = 2
NS = 16
NW = NC * NS
S_PER_W = SEQ // NW
NQ = BATCH // 2
LANES = 16
DCHUNKS = DIM // LANES


def _emb_body(x_hbm, tok_hbm, pos_hbm, out_hbm, idx_v, tok_v, pos_v,
              idx_sem, pos_sem, gat_sem, st_sem):
    wid = lax.axis_index("s") * NC + lax.axis_index("c")
    s0 = wid * S_PER_W

    idx_cps = [
        pltpu.async_copy(x_hbm.at[wid, q], idx_v.at[q], idx_sem)
        for q in range(NQ)
    ]
    pos_cp = pltpu.async_copy(pos_hbm.at[pl.ds(s0, S_PER_W)], pos_v, pos_sem)

    gathers = [None] * NQ
    for q in range(NQ):
        idx_cps[q].wait()
        gathers[q] = pltpu.async_copy(
            tok_hbm.at[idx_v.at[q]], tok_v.at[q], gat_sem)
    pos_cp.wait()

    stores = []
    for q in range(NQ):
        gathers[q].wait()

        @plsc.parallel_loop(0, S_PER_W)
        def _add(r):
            for c in range(DCHUNKS):
                sl = pl.ds(c * LANES, LANES)
                p = pos_v[r, sl]
                plsc.addupdate(tok_v.at[q, r, sl], p)
                plsc.addupdate(tok_v.at[q, S_PER_W + r, sl], p)

        for i in range(2):
            b = 2 * q + i
            stores.append(pltpu.async_copy(
                tok_v.at[q, pl.ds(i * S_PER_W, S_PER_W)],
                out_hbm.at[pl.ds(b * SEQ + s0, S_PER_W)], st_sem))
    for st in stores:
        st.wait()


def kernel(x, token_table, pos_table):
    B, S = x.shape
    xr = (x.astype(jnp.int32)
          .reshape(NQ, 2, NW, S_PER_W)
          .transpose(2, 0, 1, 3)
          .reshape(NW, NQ, 2 * S_PER_W))
    call = pl.kernel(
        _emb_body,
        out_type=jax.ShapeDtypeStruct((B * S, DIM), jnp.float32),
        mesh=plsc.VectorSubcoreMesh(core_axis_name="c", subcore_axis_name="s"),
        scratch_types=[
            pltpu.VMEM((NQ, 2 * S_PER_W), jnp.int32),
            pltpu.VMEM((NQ, 2 * S_PER_W, DIM), jnp.float32),
            pltpu.VMEM((S_PER_W, DIM), jnp.float32),
            pltpu.SemaphoreType.DMA,
            pltpu.SemaphoreType.DMA,
            pltpu.SemaphoreType.DMA,
            pltpu.SemaphoreType.DMA,
        ],
    )
    out = call(xr, token_table, pos_table)
    return out.reshape(B, S, DIM)
